# baseline (speedup 1.0000x reference)
import jax
import jax.numpy as jnp
from jax.experimental import pallas as pl
from jax.experimental.pallas import tpu as pltpu

EPS = 0.3


def _relu_linear_kernel(h_ref, w_ref, b_ref, o_ref):
    o_ref[...] = jax.nn.relu(
        jax.lax.dot_general(h_ref[...], w_ref[...], (((1,), (1,)), ((), ())),
                            preferred_element_type=jnp.float32) + b_ref[...])


def _nd(d):
    return jnp.where(d > 0, jax.lax.rsqrt(d), 0.0)


def _pack_kernel(ah_ref, at_ref, code_ref, dh_ref, dt_ref):
    r = pl.program_id(0)
    nr = pl.num_programs(0)
    ah = ah_ref[...]
    at = at_ref[...]
    code_ref[...] = (ah + 2.0 * at).astype(jnp.int8)

    @pl.when(r == 0)
    def _():
        dh_ref[...] = jnp.zeros_like(dh_ref)
        dt_ref[...] = jnp.zeros_like(dt_ref)

    dh_ref[...] += jnp.sum(ah, axis=0, keepdims=True)
    dt_ref[...] += jnp.sum(at, axis=0, keepdims=True)

    @pl.when(r == nr - 1)
    def _():
        dh_ref[...] = _nd(dh_ref[...])
        dt_ref[...] = _nd(dt_ref[...])


def _gate_kernel(x_ref, gw_ref, gb_ref, ta_ref, tb_ref):
    gw = gw_ref[...]
    hid = x_ref.shape[1]
    gwa = gw[:, :hid]
    gwb = gw[:, hid:]
    x = x_ref[...]
    a = jnp.sum(x * gwa, axis=1, keepdims=True)
    b = jnp.sum(x * gwb, axis=1, keepdims=True)
    ta_ref[...] = jnp.tanh(a + gb_ref[0, 0])
    tb_ref[...] = jnp.tanh(b)


def _fa_kernel(code_ref, ta_ref, tb_ref, ndhr_ref, ndhc_ref, ndtr_ref,
               ndtc_ref, x_ref, raw_ref, o_ref):
    r = pl.program_id(0)

    ta = ta_ref[...]
    tb = tb_ref[...]
    t = (ta + tb) / (1.0 + ta * tb)
    wh = ndhr_ref[...] * ndhc_ref[...]
    wt = ndtr_ref[...] * ndtc_ref[...]
    codef = code_ref[...].astype(jnp.float32)
    at = jnp.where(codef >= 2.0, 1.0, 0.0)
    ah = codef - 2.0 * at
    w = ((0.5 * t) * (ah * wh + at * wt)).astype(jnp.bfloat16)
    p = jax.lax.dot_general(w, x_ref[...].astype(jnp.bfloat16),
                            (((0,), (0,)), ((), ())),
                            preferred_element_type=jnp.float32)

    @pl.when(r == 0)
    def _():
        o_ref[...] = EPS * raw_ref[...]

    o_ref[...] += p


def _head_kernel(x_ref, w_ref, b_ref, o_ref):
    l = jax.lax.dot_general(x_ref[...], w_ref[...], (((1,), (1,)), ((), ())),
                            preferred_element_type=jnp.float32) + b_ref[...]
    m = jnp.max(l, axis=1, keepdims=True)
    o_ref[...] = l - m - jnp.log(jnp.sum(jnp.exp(l - m), axis=1, keepdims=True))


def kernel(h, adj_hom, adj_het, t1_w, t1_b, gate_w_0, gate_b_0, gate_w_1,
           gate_b_1, t2_w, t2_b):
    n, feat = h.shape
    hid = t1_w.shape[0]
    cls = t2_w.shape[0]
    f32 = jnp.float32

    blk = 1000 if n % 1000 == 0 else n
    nb = n // blk
    rblk = 80 if n % 80 == 0 else n
    nrb = n // rblk

    x0 = pl.pallas_call(
        _relu_linear_kernel,
        grid=(nb,),
        in_specs=[
            pl.BlockSpec((blk, feat), lambda i: (i, 0)),
            pl.BlockSpec((hid, feat), lambda i: (0, 0)),
            pl.BlockSpec((1, hid), lambda i: (0, 0)),
        ],
        out_specs=pl.BlockSpec((blk, hid), lambda i: (i, 0)),
        out_shape=jax.ShapeDtypeStruct((n, hid), f32),
    )(h, t1_w, t1_b.reshape(1, hid))

    code, ndh, ndt = pl.pallas_call(
        _pack_kernel,
        grid=(nrb,),
        in_specs=[
            pl.BlockSpec((rblk, n), lambda r: (r, 0)),
            pl.BlockSpec((rblk, n), lambda r: (r, 0)),
        ],
        out_specs=[
            pl.BlockSpec((rblk, n), lambda r: (r, 0)),
            pl.BlockSpec((1, n), lambda r: (0, 0)),
            pl.BlockSpec((1, n), lambda r: (0, 0)),
        ],
        out_shape=[
            jax.ShapeDtypeStruct((n, n), jnp.int8),
            jax.ShapeDtypeStruct((1, n), f32),
            jax.ShapeDtypeStruct((1, n), f32),
        ],
        compiler_params=pltpu.CompilerParams(
            dimension_semantics=("arbitrary",)),
    )(adj_hom, adj_het)

    ndh_c = ndh
    ndt_c = ndt
    ndh_r = ndh.reshape(n, 1)
    ndt_r = ndt.reshape(n, 1)

    gate_fn = pl.pallas_call(
        _gate_kernel,
        grid=(nb,),
        in_specs=[
            pl.BlockSpec((blk, hid), lambda i: (i, 0)),
            pl.BlockSpec((1, 2 * hid), lambda i: (0, 0)),
            pl.BlockSpec((1, 1), lambda i: (0, 0)),
        ],
        out_specs=[
            pl.BlockSpec((blk, 1), lambda i: (i, 0)),
            pl.BlockSpec((blk, 1), lambda i: (i, 0)),
        ],
        out_shape=[
            jax.ShapeDtypeStruct((n, 1), f32),
            jax.ShapeDtypeStruct((n, 1), f32),
        ],
    )

    fa_fn = pl.pallas_call(
        _fa_kernel,
        grid=(nrb,),
        in_specs=[
            pl.BlockSpec((rblk, n), lambda r: (r, 0)),
            pl.BlockSpec((rblk, 1), lambda r: (r, 0)),
            pl.BlockSpec((1, n), lambda r: (0, 0)),
            pl.BlockSpec((rblk, 1), lambda r: (r, 0)),
            pl.BlockSpec((1, n), lambda r: (0, 0)),
            pl.BlockSpec((rblk, 1), lambda r: (r, 0)),
            pl.BlockSpec((1, n), lambda r: (0, 0)),
            pl.BlockSpec((rblk, hid), lambda r: (r, 0)),
            pl.BlockSpec((n, hid), lambda r: (0, 0)),
        ],
        out_specs=pl.BlockSpec((n, hid), lambda r: (0, 0)),
        out_shape=jax.ShapeDtypeStruct((n, hid), f32),
        compiler_params=pltpu.CompilerParams(
            dimension_semantics=("arbitrary",)),
    )

    x = x0
    for gw, gb in ((gate_w_0, gate_b_0), (gate_w_1, gate_b_1)):
        ta, tb = gate_fn(x, gw, gb.reshape(1, 1))
        x = fa_fn(code, ta, tb.reshape(1, n), ndh_r, ndh_c, ndt_r,
                  ndt_c, x, x0)

    out = pl.pallas_call(
        _head_kernel,
        grid=(nb,),
        in_specs=[
            pl.BlockSpec((blk, hid), lambda i: (i, 0)),
            pl.BlockSpec((cls, hid), lambda i: (0, 0)),
            pl.BlockSpec((1, cls), lambda i: (0, 0)),
        ],
        out_specs=pl.BlockSpec((blk, cls), lambda i: (i, 0)),
        out_shape=jax.ShapeDtypeStruct((n, cls), f32),
    )(x, t2_w, t2_b.reshape(1, cls))

    return out


# speedup vs baseline: 1.5871x; 1.5871x over previous
import jax
import jax.numpy as jnp
from jax.experimental import pallas as pl
from jax.experimental.pallas import tpu as pltpu

EPS = 0.3


def _relu_linear_kernel(h_ref, w_ref, b_ref, o_ref):
    o_ref[...] = jax.nn.relu(
        jax.lax.dot_general(h_ref[...], w_ref[...], (((1,), (1,)), ((), ())),
                            preferred_element_type=jnp.float32) + b_ref[...])


def _nd(d):
    return jnp.where(d > 0, jax.lax.rsqrt(d), 0.0)


def _pack_kernel(ah_ref, at_ref, code_ref, dh_ref, dt_ref):
    r = pl.program_id(0)
    nr = pl.num_programs(0)
    ah = ah_ref[...]
    at = at_ref[...]
    code_ref[...] = (ah + 2.0 * at).astype(jnp.int8)

    @pl.when(r == 0)
    def _():
        dh_ref[...] = jnp.zeros_like(dh_ref)
        dt_ref[...] = jnp.zeros_like(dt_ref)

    dh_ref[...] += jnp.sum(ah, axis=0, keepdims=True)
    dt_ref[...] += jnp.sum(at, axis=0, keepdims=True)

    @pl.when(r == nr - 1)
    def _():
        dh_ref[...] = _nd(dh_ref[...])
        dt_ref[...] = _nd(dt_ref[...])


def _gate_kernel(x_ref, gw_ref, gb_ref, ta_ref, tb_ref):
    gw = gw_ref[...]
    hid = x_ref.shape[1]
    gwa = gw[:, :hid]
    gwb = gw[:, hid:]
    x = x_ref[...]
    a = jnp.sum(x * gwa, axis=1, keepdims=True)
    b = jnp.sum(x * gwb, axis=1, keepdims=True)
    ta_ref[...] = jnp.tanh(a + gb_ref[0, 0])
    tb_ref[...] = jnp.tanh(b)


def _gate_t(ta_ref, tb_ref):
    ta = ta_ref[...]
    tb = tb_ref[...]
    return (ta + tb) / (1.0 + ta * tb)


def _fa1_kernel(code_ref, ta_ref, tb_ref, ndhr_ref, ndhc_ref, ndtr_ref,
                ndtc_ref, x_ref, raw_ref, o_ref, m_ref):
    r = pl.program_id(0)

    t = _gate_t(ta_ref, tb_ref)
    wh = (0.5 * ndhr_ref[...]) * ndhc_ref[...]
    wt = (0.5 * ndtr_ref[...]) * ndtc_ref[...]
    codef = code_ref[...].astype(jnp.float32)
    at = jnp.where(codef >= 2.0, 1.0, 0.0)
    ah = codef - (at + at)
    m = ah * wh + at * wt
    m_ref[...] = m.astype(jnp.bfloat16)
    w = (t * m).astype(jnp.bfloat16)
    p = jax.lax.dot_general(w, x_ref[...].astype(jnp.bfloat16),
                            (((0,), (0,)), ((), ())),
                            preferred_element_type=jnp.float32)

    @pl.when(r == 0)
    def _():
        o_ref[...] = EPS * raw_ref[...]

    o_ref[...] += p


def _fa2_kernel(m_ref, ta_ref, tb_ref, x_ref, raw_ref, o_ref):
    r = pl.program_id(0)

    t = _gate_t(ta_ref, tb_ref)
    w = (t * m_ref[...].astype(jnp.float32)).astype(jnp.bfloat16)
    p = jax.lax.dot_general(w, x_ref[...].astype(jnp.bfloat16),
                            (((0,), (0,)), ((), ())),
                            preferred_element_type=jnp.float32)

    @pl.when(r == 0)
    def _():
        o_ref[...] = EPS * raw_ref[...]

    o_ref[...] += p


def _head_kernel(x_ref, w_ref, b_ref, o_ref):
    l = jax.lax.dot_general(x_ref[...], w_ref[...], (((1,), (1,)), ((), ())),
                            preferred_element_type=jnp.float32) + b_ref[...]
    m = jnp.max(l, axis=1, keepdims=True)
    o_ref[...] = l - m - jnp.log(jnp.sum(jnp.exp(l - m), axis=1, keepdims=True))


def kernel(h, adj_hom, adj_het, t1_w, t1_b, gate_w_0, gate_b_0, gate_w_1,
           gate_b_1, t2_w, t2_b):
    n, feat = h.shape
    hid = t1_w.shape[0]
    cls = t2_w.shape[0]
    f32 = jnp.float32

    blk = 1000 if n % 1000 == 0 else n
    nb = n // blk
    pblk = 200 if n % 200 == 0 else n
    npb = n // pblk
    rblk = 400 if n % 400 == 0 else n
    nrb = n // rblk

    x0 = pl.pallas_call(
        _relu_linear_kernel,
        grid=(nb,),
        in_specs=[
            pl.BlockSpec((blk, feat), lambda i: (i, 0)),
            pl.BlockSpec((hid, feat), lambda i: (0, 0)),
            pl.BlockSpec((1, hid), lambda i: (0, 0)),
        ],
        out_specs=pl.BlockSpec((blk, hid), lambda i: (i, 0)),
        out_shape=jax.ShapeDtypeStruct((n, hid), f32),
    )(h, t1_w, t1_b.reshape(1, hid))

    code, ndh, ndt = pl.pallas_call(
        _pack_kernel,
        grid=(npb,),
        in_specs=[
            pl.BlockSpec((pblk, n), lambda r: (r, 0)),
            pl.BlockSpec((pblk, n), lambda r: (r, 0)),
        ],
        out_specs=[
            pl.BlockSpec((pblk, n), lambda r: (r, 0)),
            pl.BlockSpec((1, n), lambda r: (0, 0)),
            pl.BlockSpec((1, n), lambda r: (0, 0)),
        ],
        out_shape=[
            jax.ShapeDtypeStruct((n, n), jnp.int8),
            jax.ShapeDtypeStruct((1, n), f32),
            jax.ShapeDtypeStruct((1, n), f32),
        ],
        compiler_params=pltpu.CompilerParams(
            dimension_semantics=("arbitrary",)),
    )(adj_hom, adj_het)

    ndh_c = ndh
    ndt_c = ndt
    ndh_r = ndh.reshape(n, 1)
    ndt_r = ndt.reshape(n, 1)

    gate_fn = pl.pallas_call(
        _gate_kernel,
        grid=(nb,),
        in_specs=[
            pl.BlockSpec((blk, hid), lambda i: (i, 0)),
            pl.BlockSpec((1, 2 * hid), lambda i: (0, 0)),
            pl.BlockSpec((1, 1), lambda i: (0, 0)),
        ],
        out_specs=[
            pl.BlockSpec((blk, 1), lambda i: (i, 0)),
            pl.BlockSpec((blk, 1), lambda i: (i, 0)),
        ],
        out_shape=[
            jax.ShapeDtypeStruct((n, 1), f32),
            jax.ShapeDtypeStruct((n, 1), f32),
        ],
    )

    fa1_fn = pl.pallas_call(
        _fa1_kernel,
        grid=(nrb,),
        in_specs=[
            pl.BlockSpec((rblk, n), lambda r: (r, 0)),
            pl.BlockSpec((rblk, 1), lambda r: (r, 0)),
            pl.BlockSpec((1, n), lambda r: (0, 0)),
            pl.BlockSpec((rblk, 1), lambda r: (r, 0)),
            pl.BlockSpec((1, n), lambda r: (0, 0)),
            pl.BlockSpec((rblk, 1), lambda r: (r, 0)),
            pl.BlockSpec((1, n), lambda r: (0, 0)),
            pl.BlockSpec((rblk, hid), lambda r: (r, 0)),
            pl.BlockSpec((n, hid), lambda r: (0, 0)),
        ],
        out_specs=[
            pl.BlockSpec((n, hid), lambda r: (0, 0)),
            pl.BlockSpec((rblk, n), lambda r: (r, 0)),
        ],
        out_shape=[
            jax.ShapeDtypeStruct((n, hid), f32),
            jax.ShapeDtypeStruct((n, n), jnp.bfloat16),
        ],
        compiler_params=pltpu.CompilerParams(
            dimension_semantics=("arbitrary",)),
    )

    fa2_fn = pl.pallas_call(
        _fa2_kernel,
        grid=(nrb,),
        in_specs=[
            pl.BlockSpec((rblk, n), lambda r: (r, 0)),
            pl.BlockSpec((rblk, 1), lambda r: (r, 0)),
            pl.BlockSpec((1, n), lambda r: (0, 0)),
            pl.BlockSpec((rblk, hid), lambda r: (r, 0)),
            pl.BlockSpec((n, hid), lambda r: (0, 0)),
        ],
        out_specs=pl.BlockSpec((n, hid), lambda r: (0, 0)),
        out_shape=jax.ShapeDtypeStruct((n, hid), f32),
        compiler_params=pltpu.CompilerParams(
            dimension_semantics=("arbitrary",)),
    )

    ta, tb = gate_fn(x0, gate_w_0, gate_b_0.reshape(1, 1))
    x1, m = fa1_fn(code, ta, tb.reshape(1, n), ndh_r, ndh_c, ndt_r,
                   ndt_c, x0, x0)
    ta, tb = gate_fn(x1, gate_w_1, gate_b_1.reshape(1, 1))
    x = fa2_fn(m, ta, tb.reshape(1, n), x1, x0)

    out = pl.pallas_call(
        _head_kernel,
        grid=(nb,),
        in_specs=[
            pl.BlockSpec((blk, hid), lambda i: (i, 0)),
            pl.BlockSpec((cls, hid), lambda i: (0, 0)),
            pl.BlockSpec((1, cls), lambda i: (0, 0)),
        ],
        out_specs=pl.BlockSpec((blk, cls), lambda i: (i, 0)),
        out_shape=jax.ShapeDtypeStruct((n, cls), f32),
    )(x, t2_w, t2_b.reshape(1, cls))

    return out


# speedup vs baseline: 1.6627x; 1.0477x over previous
import jax
import jax.numpy as jnp
from jax.experimental import pallas as pl
from jax.experimental.pallas import tpu as pltpu

EPS = 0.3


def _relu_linear_kernel(h_ref, w_ref, b_ref, o_ref):
    o_ref[...] = jax.nn.relu(
        jax.lax.dot_general(h_ref[...], w_ref[...], (((1,), (1,)), ((), ())),
                            preferred_element_type=jnp.float32) + b_ref[...])


def _nd(d):
    return jnp.where(d > 0, jax.lax.rsqrt(d), 0.0)


def _pack_kernel(ah_ref, at_ref, code_ref, dh_ref, dt_ref):
    r = pl.program_id(0)
    nr = pl.num_programs(0)
    ah = ah_ref[...]
    at = at_ref[...]
    code_ref[...] = (ah + 2.0 * at).astype(jnp.int8)

    @pl.when(r == 0)
    def _():
        dh_ref[...] = jnp.zeros_like(dh_ref)
        dt_ref[...] = jnp.zeros_like(dt_ref)

    dh_ref[...] += jnp.sum(ah, axis=0, keepdims=True)
    dt_ref[...] += jnp.sum(at, axis=0, keepdims=True)

    @pl.when(r == nr - 1)
    def _():
        dh_ref[...] = _nd(dh_ref[...])
        dt_ref[...] = _nd(dt_ref[...])


def _gate_kernel(x_ref, gw_ref, gb_ref, ta_ref, tb_ref):
    gw = gw_ref[...]
    hid = x_ref.shape[1]
    gwa = gw[:, :hid]
    gwb = gw[:, hid:]
    x = x_ref[...]
    a = jnp.sum(x * gwa, axis=1, keepdims=True)
    b = jnp.sum(x * gwb, axis=1, keepdims=True)
    ta_ref[...] = a + gb_ref[0, 0]
    tb_ref[...] = b


def _gate_t(ta_ref, tb_ref):
    ta = ta_ref[...]
    tb = tb_ref[...]
    return jnp.tanh(ta + tb)


def _fa_matmul(w, x_ref, raw_ref, o_ref, r):
    xt = x_ref[...].T.astype(jnp.bfloat16)
    p = jax.lax.dot_general(xt, w, (((1,), (0,)), ((), ())),
                            preferred_element_type=jnp.float32)

    @pl.when(r == 0)
    def _():
        o_ref[...] = EPS * raw_ref[...]

    o_ref[...] += p


def _fa1_kernel(code_ref, ta_ref, tb_ref, ndhr_ref, ndhc_ref, ndtr_ref,
                ndtc_ref, x_ref, raw_ref, o_ref, m_ref):
    t = _gate_t(ta_ref, tb_ref)
    wh = (0.5 * ndhr_ref[...]) * ndhc_ref[...]
    wt = (0.5 * ndtr_ref[...]) * ndtc_ref[...]
    codef = code_ref[...].astype(jnp.float32)
    at = jnp.where(codef >= 2.0, 1.0, 0.0)
    ah = codef - (at + at)
    m = ah * wh + at * wt
    m_ref[...] = m.astype(jnp.bfloat16)
    w = (t * m).astype(jnp.bfloat16)
    _fa_matmul(w, x_ref, raw_ref, o_ref, pl.program_id(0))


def _fa2_kernel(m_ref, ta_ref, tb_ref, x_ref, raw_ref, o_ref):
    t = _gate_t(ta_ref, tb_ref)
    w = (t * m_ref[...].astype(jnp.float32)).astype(jnp.bfloat16)
    _fa_matmul(w, x_ref, raw_ref, o_ref, pl.program_id(0))


def _head_kernel(x_ref, w_ref, b_ref, o_ref):
    l = jax.lax.dot_general(x_ref[...], w_ref[...], (((1,), (1,)), ((), ())),
                            preferred_element_type=jnp.float32) + b_ref[...]
    m = jnp.max(l, axis=1, keepdims=True)
    o_ref[...] = l - m - jnp.log(jnp.sum(jnp.exp(l - m), axis=1, keepdims=True))


def kernel(h, adj_hom, adj_het, t1_w, t1_b, gate_w_0, gate_b_0, gate_w_1,
           gate_b_1, t2_w, t2_b):
    n, feat = h.shape
    hid = t1_w.shape[0]
    cls = t2_w.shape[0]
    f32 = jnp.float32

    blk = 1000 if n % 1000 == 0 else n
    nb = n // blk
    pblk = 200 if n % 200 == 0 else n
    npb = n // pblk
    rblk = 400 if n % 400 == 0 else n
    nrb = n // rblk

    x0 = pl.pallas_call(
        _relu_linear_kernel,
        grid=(nb,),
        in_specs=[
            pl.BlockSpec((blk, feat), lambda i: (i, 0)),
            pl.BlockSpec((hid, feat), lambda i: (0, 0)),
            pl.BlockSpec((1, hid), lambda i: (0, 0)),
        ],
        out_specs=pl.BlockSpec((blk, hid), lambda i: (i, 0)),
        out_shape=jax.ShapeDtypeStruct((n, hid), f32),
    )(h, t1_w, t1_b.reshape(1, hid))

    code, ndh, ndt = pl.pallas_call(
        _pack_kernel,
        grid=(npb,),
        in_specs=[
            pl.BlockSpec((pblk, n), lambda r: (r, 0)),
            pl.BlockSpec((pblk, n), lambda r: (r, 0)),
        ],
        out_specs=[
            pl.BlockSpec((pblk, n), lambda r: (r, 0)),
            pl.BlockSpec((1, n), lambda r: (0, 0)),
            pl.BlockSpec((1, n), lambda r: (0, 0)),
        ],
        out_shape=[
            jax.ShapeDtypeStruct((n, n), jnp.int8),
            jax.ShapeDtypeStruct((1, n), f32),
            jax.ShapeDtypeStruct((1, n), f32),
        ],
        compiler_params=pltpu.CompilerParams(
            dimension_semantics=("arbitrary",)),
    )(adj_hom, adj_het)

    ndh_c = ndh
    ndt_c = ndt
    ndh_r = ndh.reshape(n, 1)
    ndt_r = ndt.reshape(n, 1)

    gate_fn = pl.pallas_call(
        _gate_kernel,
        grid=(nb,),
        in_specs=[
            pl.BlockSpec((blk, hid), lambda i: (i, 0)),
            pl.BlockSpec((1, 2 * hid), lambda i: (0, 0)),
            pl.BlockSpec((1, 1), lambda i: (0, 0)),
        ],
        out_specs=[
            pl.BlockSpec((blk, 1), lambda i: (i, 0)),
            pl.BlockSpec((blk, 1), lambda i: (i, 0)),
        ],
        out_shape=[
            jax.ShapeDtypeStruct((n, 1), f32),
            jax.ShapeDtypeStruct((n, 1), f32),
        ],
    )

    fa1_fn = pl.pallas_call(
        _fa1_kernel,
        grid=(nrb,),
        in_specs=[
            pl.BlockSpec((rblk, n), lambda r: (r, 0)),
            pl.BlockSpec((rblk, 1), lambda r: (r, 0)),
            pl.BlockSpec((1, n), lambda r: (0, 0)),
            pl.BlockSpec((rblk, 1), lambda r: (r, 0)),
            pl.BlockSpec((1, n), lambda r: (0, 0)),
            pl.BlockSpec((rblk, 1), lambda r: (r, 0)),
            pl.BlockSpec((1, n), lambda r: (0, 0)),
            pl.BlockSpec((rblk, hid), lambda r: (r, 0)),
            pl.BlockSpec((hid, n), lambda r: (0, 0)),
        ],
        out_specs=[
            pl.BlockSpec((hid, n), lambda r: (0, 0)),
            pl.BlockSpec((rblk, n), lambda r: (r, 0)),
        ],
        out_shape=[
            jax.ShapeDtypeStruct((hid, n), f32),
            jax.ShapeDtypeStruct((n, n), jnp.bfloat16),
        ],
        compiler_params=pltpu.CompilerParams(
            dimension_semantics=("arbitrary",)),
    )

    fa2_fn = pl.pallas_call(
        _fa2_kernel,
        grid=(nrb,),
        in_specs=[
            pl.BlockSpec((rblk, n), lambda r: (r, 0)),
            pl.BlockSpec((rblk, 1), lambda r: (r, 0)),
            pl.BlockSpec((1, n), lambda r: (0, 0)),
            pl.BlockSpec((rblk, hid), lambda r: (r, 0)),
            pl.BlockSpec((hid, n), lambda r: (0, 0)),
        ],
        out_specs=pl.BlockSpec((hid, n), lambda r: (0, 0)),
        out_shape=jax.ShapeDtypeStruct((hid, n), f32),
        compiler_params=pltpu.CompilerParams(
            dimension_semantics=("arbitrary",)),
    )

    raw_t = x0.T
    ta, tb = gate_fn(x0, gate_w_0, gate_b_0.reshape(1, 1))
    x1_t, m = fa1_fn(code, ta, tb.reshape(1, n), ndh_r, ndh_c, ndt_r,
                     ndt_c, x0, raw_t)
    x1 = x1_t.T
    ta, tb = gate_fn(x1, gate_w_1, gate_b_1.reshape(1, 1))
    x = fa2_fn(m, ta, tb.reshape(1, n), x1, raw_t).T

    out = pl.pallas_call(
        _head_kernel,
        grid=(nb,),
        in_specs=[
            pl.BlockSpec((blk, hid), lambda i: (i, 0)),
            pl.BlockSpec((cls, hid), lambda i: (0, 0)),
            pl.BlockSpec((1, cls), lambda i: (0, 0)),
        ],
        out_specs=pl.BlockSpec((blk, cls), lambda i: (i, 0)),
        out_shape=jax.ShapeDtypeStruct((n, cls), f32),
    )(x, t2_w, t2_b.reshape(1, cls))

    return out


# speedup vs baseline: 1.9479x; 1.1715x over previous
import jax
import jax.numpy as jnp
from jax.experimental import pallas as pl
from jax.experimental.pallas import tpu as pltpu

EPS = 0.3


def _relu_linear_kernel(h_ref, w_ref, b_ref, o_ref):
    o_ref[...] = jax.nn.relu(
        jax.lax.dot_general(h_ref[...], w_ref[...], (((1,), (1,)), ((), ())),
                            preferred_element_type=jnp.float32) + b_ref[...])


def _nd(d):
    return jnp.where(d > 0, jax.lax.rsqrt(d), 0.0)


def _pack_kernel(ah_ref, at_ref, code_ref, dh_ref, dt_ref):
    r = pl.program_id(0)
    nr = pl.num_programs(0)
    ah = ah_ref[...]
    at = at_ref[...]
    code_ref[...] = (ah + 2.0 * at).astype(jnp.int8)

    @pl.when(r == 0)
    def _():
        dh_ref[...] = jnp.zeros_like(dh_ref)
        dt_ref[...] = jnp.zeros_like(dt_ref)

    dh_ref[...] += jnp.sum(ah, axis=0, keepdims=True)
    dt_ref[...] += jnp.sum(at, axis=0, keepdims=True)

    @pl.when(r == nr - 1)
    def _():
        dh_ref[...] = _nd(dh_ref[...])
        dt_ref[...] = _nd(dt_ref[...])


def _gate_kernel(x_ref, gw_ref, gb_ref, ta_ref, tb_ref):
    gw = gw_ref[...]
    hid = x_ref.shape[1]
    gwa = gw[:, :hid]
    gwb = gw[:, hid:]
    x = x_ref[...]
    a = jnp.sum(x * gwa, axis=1, keepdims=True)
    b = jnp.sum(x * gwb, axis=1, keepdims=True)
    ta_ref[...] = a + gb_ref[0, 0]
    tb_ref[...] = b


def _gate_t(ta_ref, tb_ref):
    ta = ta_ref[...].astype(jnp.bfloat16)
    tb = tb_ref[...].astype(jnp.bfloat16)
    return jnp.tanh(ta + tb)


def _fa_matmul(w, x_ref, raw_ref, o_ref, r):
    xt = x_ref[...].T.astype(jnp.bfloat16)
    p = jax.lax.dot_general(xt, w, (((1,), (0,)), ((), ())),
                            preferred_element_type=jnp.float32)

    @pl.when(r == 0)
    def _():
        o_ref[...] = EPS * raw_ref[...]

    o_ref[...] += p


def _fa1_kernel(code_ref, ta_ref, tb_ref, ndhr_ref, ndhc_ref, ndtr_ref,
                ndtc_ref, x_ref, raw_ref, o_ref, m_ref):
    bf = jnp.bfloat16
    t = _gate_t(ta_ref, tb_ref)
    wh = ((0.5 * ndhr_ref[...]).astype(bf)) * ndhc_ref[...].astype(bf)
    wt = ((0.5 * ndtr_ref[...]).astype(bf)) * ndtc_ref[...].astype(bf)
    codef = code_ref[...].astype(bf)
    at = jnp.where(codef >= 2, jnp.ones((), bf), jnp.zeros((), bf))
    ah = codef - (at + at)
    m = ah * wh + at * wt
    m_ref[...] = m
    w = t * m
    _fa_matmul(w, x_ref, raw_ref, o_ref, pl.program_id(0))


def _fa2_kernel(m_ref, ta_ref, tb_ref, x_ref, raw_ref, o_ref):
    t = _gate_t(ta_ref, tb_ref)
    w = t * m_ref[...]
    _fa_matmul(w, x_ref, raw_ref, o_ref, pl.program_id(0))


def _head_kernel(x_ref, w_ref, b_ref, o_ref):
    l = jax.lax.dot_general(x_ref[...], w_ref[...], (((1,), (1,)), ((), ())),
                            preferred_element_type=jnp.float32) + b_ref[...]
    m = jnp.max(l, axis=1, keepdims=True)
    o_ref[...] = l - m - jnp.log(jnp.sum(jnp.exp(l - m), axis=1, keepdims=True))


def kernel(h, adj_hom, adj_het, t1_w, t1_b, gate_w_0, gate_b_0, gate_w_1,
           gate_b_1, t2_w, t2_b):
    n, feat = h.shape
    hid = t1_w.shape[0]
    cls = t2_w.shape[0]
    f32 = jnp.float32

    blk = 1000 if n % 1000 == 0 else n
    nb = n // blk
    pblk = 200 if n % 200 == 0 else n
    npb = n // pblk
    rblk = 400 if n % 400 == 0 else n
    nrb = n // rblk

    x0 = pl.pallas_call(
        _relu_linear_kernel,
        grid=(nb,),
        in_specs=[
            pl.BlockSpec((blk, feat), lambda i: (i, 0)),
            pl.BlockSpec((hid, feat), lambda i: (0, 0)),
            pl.BlockSpec((1, hid), lambda i: (0, 0)),
        ],
        out_specs=pl.BlockSpec((blk, hid), lambda i: (i, 0)),
        out_shape=jax.ShapeDtypeStruct((n, hid), f32),
    )(h, t1_w, t1_b.reshape(1, hid))

    code, ndh, ndt = pl.pallas_call(
        _pack_kernel,
        grid=(npb,),
        in_specs=[
            pl.BlockSpec((pblk, n), lambda r: (r, 0)),
            pl.BlockSpec((pblk, n), lambda r: (r, 0)),
        ],
        out_specs=[
            pl.BlockSpec((pblk, n), lambda r: (r, 0)),
            pl.BlockSpec((1, n), lambda r: (0, 0)),
            pl.BlockSpec((1, n), lambda r: (0, 0)),
        ],
        out_shape=[
            jax.ShapeDtypeStruct((n, n), jnp.int8),
            jax.ShapeDtypeStruct((1, n), f32),
            jax.ShapeDtypeStruct((1, n), f32),
        ],
        compiler_params=pltpu.CompilerParams(
            dimension_semantics=("arbitrary",)),
    )(adj_hom, adj_het)

    ndh_c = ndh
    ndt_c = ndt
    ndh_r = ndh.reshape(n, 1)
    ndt_r = ndt.reshape(n, 1)

    gate_fn = pl.pallas_call(
        _gate_kernel,
        grid=(nb,),
        in_specs=[
            pl.BlockSpec((blk, hid), lambda i: (i, 0)),
            pl.BlockSpec((1, 2 * hid), lambda i: (0, 0)),
            pl.BlockSpec((1, 1), lambda i: (0, 0)),
        ],
        out_specs=[
            pl.BlockSpec((blk, 1), lambda i: (i, 0)),
            pl.BlockSpec((blk, 1), lambda i: (i, 0)),
        ],
        out_shape=[
            jax.ShapeDtypeStruct((n, 1), f32),
            jax.ShapeDtypeStruct((n, 1), f32),
        ],
    )

    fa1_fn = pl.pallas_call(
        _fa1_kernel,
        grid=(nrb,),
        in_specs=[
            pl.BlockSpec((rblk, n), lambda r: (r, 0)),
            pl.BlockSpec((rblk, 1), lambda r: (r, 0)),
            pl.BlockSpec((1, n), lambda r: (0, 0)),
            pl.BlockSpec((rblk, 1), lambda r: (r, 0)),
            pl.BlockSpec((1, n), lambda r: (0, 0)),
            pl.BlockSpec((rblk, 1), lambda r: (r, 0)),
            pl.BlockSpec((1, n), lambda r: (0, 0)),
            pl.BlockSpec((rblk, hid), lambda r: (r, 0)),
            pl.BlockSpec((hid, n), lambda r: (0, 0)),
        ],
        out_specs=[
            pl.BlockSpec((hid, n), lambda r: (0, 0)),
            pl.BlockSpec((rblk, n), lambda r: (r, 0)),
        ],
        out_shape=[
            jax.ShapeDtypeStruct((hid, n), f32),
            jax.ShapeDtypeStruct((n, n), jnp.bfloat16),
        ],
        compiler_params=pltpu.CompilerParams(
            dimension_semantics=("arbitrary",)),
    )

    fa2_fn = pl.pallas_call(
        _fa2_kernel,
        grid=(nrb,),
        in_specs=[
            pl.BlockSpec((rblk, n), lambda r: (r, 0)),
            pl.BlockSpec((rblk, 1), lambda r: (r, 0)),
            pl.BlockSpec((1, n), lambda r: (0, 0)),
            pl.BlockSpec((rblk, hid), lambda r: (r, 0)),
            pl.BlockSpec((hid, n), lambda r: (0, 0)),
        ],
        out_specs=pl.BlockSpec((hid, n), lambda r: (0, 0)),
        out_shape=jax.ShapeDtypeStruct((hid, n), f32),
        compiler_params=pltpu.CompilerParams(
            dimension_semantics=("arbitrary",)),
    )

    raw_t = x0.T
    ta, tb = gate_fn(x0, gate_w_0, gate_b_0.reshape(1, 1))
    x1_t, m = fa1_fn(code, ta, tb.reshape(1, n), ndh_r, ndh_c, ndt_r,
                     ndt_c, x0, raw_t)
    x1 = x1_t.T
    ta, tb = gate_fn(x1, gate_w_1, gate_b_1.reshape(1, 1))
    x = fa2_fn(m, ta, tb.reshape(1, n), x1, raw_t).T

    out = pl.pallas_call(
        _head_kernel,
        grid=(nb,),
        in_specs=[
            pl.BlockSpec((blk, hid), lambda i: (i, 0)),
            pl.BlockSpec((cls, hid), lambda i: (0, 0)),
            pl.BlockSpec((1, cls), lambda i: (0, 0)),
        ],
        out_specs=pl.BlockSpec((blk, cls), lambda i: (i, 0)),
        out_shape=jax.ShapeDtypeStruct((n, cls), f32),
    )(x, t2_w, t2_b.reshape(1, cls))

    return out


# speedup vs baseline: 1.9663x; 1.0095x over previous
import jax
import jax.numpy as jnp
from jax.experimental import pallas as pl
from jax.experimental.pallas import tpu as pltpu

EPS = 0.3


def _relu_linear_kernel(h_ref, w_ref, b_ref, o_ref):
    o_ref[...] = jax.nn.relu(
        jax.lax.dot_general(h_ref[...], w_ref[...], (((1,), (1,)), ((), ())),
                            preferred_element_type=jnp.float32) + b_ref[...])


def _nd(d):
    return jnp.where(d > 0, jax.lax.rsqrt(d), 0.0)


def _pack_kernel(ah_ref, at_ref, code_ref, dh_ref, dt_ref):
    r = pl.program_id(0)
    nr = pl.num_programs(0)
    ah = ah_ref[...]
    at = at_ref[...]
    code_ref[...] = (ah + 2.0 * at).astype(jnp.int8)

    @pl.when(r == 0)
    def _():
        dh_ref[...] = jnp.zeros_like(dh_ref)
        dt_ref[...] = jnp.zeros_like(dt_ref)

    dh_ref[...] += jnp.sum(ah, axis=0, keepdims=True)
    dt_ref[...] += jnp.sum(at, axis=0, keepdims=True)

    @pl.when(r == nr - 1)
    def _():
        dh_ref[...] = _nd(dh_ref[...])
        dt_ref[...] = _nd(dt_ref[...])


def _gate_kernel(x_ref, gw_ref, gb_ref, ta_ref, tb_ref):
    gw = gw_ref[...]
    hid = x_ref.shape[1]
    gwa = gw[:, :hid]
    gwb = gw[:, hid:]
    x = x_ref[...]
    a = jnp.sum(x * gwa, axis=1, keepdims=True)
    b = jnp.sum(x * gwb, axis=1, keepdims=True)
    ta_ref[...] = a + gb_ref[0, 0]
    tb_ref[...] = b


def _gate_t(ta_ref, tb_ref):
    ta = ta_ref[...].astype(jnp.bfloat16)
    tb = tb_ref[...].astype(jnp.bfloat16)
    return jnp.tanh(ta + tb)


def _fa_matmul(w, x_ref, raw_ref, o_ref, on_ref, r, nr):
    xt = x_ref[...].T.astype(jnp.bfloat16)
    p = jax.lax.dot_general(xt, w, (((1,), (0,)), ((), ())),
                            preferred_element_type=jnp.float32)

    @pl.when(r == 0)
    def _():
        o_ref[...] = EPS * raw_ref[...]

    o_ref[...] += p

    @pl.when(r == nr - 1)
    def _():
        on_ref[...] = o_ref[...].T


def _fa1_kernel(code_ref, ta_ref, tb_ref, ndhr_ref, ndhc_ref, ndtr_ref,
                ndtc_ref, x_ref, raw_ref, o_ref, m_ref, on_ref):
    bf = jnp.bfloat16
    t = _gate_t(ta_ref, tb_ref)
    wh = ((0.5 * ndhr_ref[...]).astype(bf)) * ndhc_ref[...].astype(bf)
    wt = ((0.5 * ndtr_ref[...]).astype(bf)) * ndtc_ref[...].astype(bf)
    codef = code_ref[...].astype(bf)
    at = jnp.where(codef >= 2, jnp.ones((), bf), jnp.zeros((), bf))
    ah = codef - (at + at)
    m = ah * wh + at * wt
    m_ref[...] = m
    w = t * m
    _fa_matmul(w, x_ref, raw_ref, o_ref, on_ref, pl.program_id(0),
               pl.num_programs(0))


def _fa2_kernel(m_ref, ta_ref, tb_ref, x_ref, raw_ref, o_ref, on_ref):
    t = _gate_t(ta_ref, tb_ref)
    w = t * m_ref[...]
    _fa_matmul(w, x_ref, raw_ref, o_ref, on_ref, pl.program_id(0),
               pl.num_programs(0))


def _head_kernel(x_ref, w_ref, b_ref, o_ref):
    l = jax.lax.dot_general(x_ref[...], w_ref[...], (((1,), (1,)), ((), ())),
                            preferred_element_type=jnp.float32) + b_ref[...]
    m = jnp.max(l, axis=1, keepdims=True)
    o_ref[...] = l - m - jnp.log(jnp.sum(jnp.exp(l - m), axis=1, keepdims=True))


def kernel(h, adj_hom, adj_het, t1_w, t1_b, gate_w_0, gate_b_0, gate_w_1,
           gate_b_1, t2_w, t2_b):
    n, feat = h.shape
    hid = t1_w.shape[0]
    cls = t2_w.shape[0]
    f32 = jnp.float32

    blk = 1000 if n % 1000 == 0 else n
    nb = n // blk
    pblk = 200 if n % 200 == 0 else n
    npb = n // pblk
    rblk = 400 if n % 400 == 0 else n
    nrb = n // rblk

    x0 = pl.pallas_call(
        _relu_linear_kernel,
        grid=(nb,),
        in_specs=[
            pl.BlockSpec((blk, feat), lambda i: (i, 0)),
            pl.BlockSpec((hid, feat), lambda i: (0, 0)),
            pl.BlockSpec((1, hid), lambda i: (0, 0)),
        ],
        out_specs=pl.BlockSpec((blk, hid), lambda i: (i, 0)),
        out_shape=jax.ShapeDtypeStruct((n, hid), f32),
    )(h, t1_w, t1_b.reshape(1, hid))

    code, ndh, ndt = pl.pallas_call(
        _pack_kernel,
        grid=(npb,),
        in_specs=[
            pl.BlockSpec((pblk, n), lambda r: (r, 0)),
            pl.BlockSpec((pblk, n), lambda r: (r, 0)),
        ],
        out_specs=[
            pl.BlockSpec((pblk, n), lambda r: (r, 0)),
            pl.BlockSpec((1, n), lambda r: (0, 0)),
            pl.BlockSpec((1, n), lambda r: (0, 0)),
        ],
        out_shape=[
            jax.ShapeDtypeStruct((n, n), jnp.int8),
            jax.ShapeDtypeStruct((1, n), f32),
            jax.ShapeDtypeStruct((1, n), f32),
        ],
        compiler_params=pltpu.CompilerParams(
            dimension_semantics=("arbitrary",)),
    )(adj_hom, adj_het)

    ndh_c = ndh
    ndt_c = ndt
    ndh_r = ndh.reshape(n, 1)
    ndt_r = ndt.reshape(n, 1)

    gate_fn = pl.pallas_call(
        _gate_kernel,
        grid=(nb,),
        in_specs=[
            pl.BlockSpec((blk, hid), lambda i: (i, 0)),
            pl.BlockSpec((1, 2 * hid), lambda i: (0, 0)),
            pl.BlockSpec((1, 1), lambda i: (0, 0)),
        ],
        out_specs=[
            pl.BlockSpec((blk, 1), lambda i: (i, 0)),
            pl.BlockSpec((blk, 1), lambda i: (i, 0)),
        ],
        out_shape=[
            jax.ShapeDtypeStruct((n, 1), f32),
            jax.ShapeDtypeStruct((n, 1), f32),
        ],
    )

    fa1_fn = pl.pallas_call(
        _fa1_kernel,
        grid=(nrb,),
        in_specs=[
            pl.BlockSpec((rblk, n), lambda r: (r, 0)),
            pl.BlockSpec((rblk, 1), lambda r: (r, 0)),
            pl.BlockSpec((1, n), lambda r: (0, 0)),
            pl.BlockSpec((rblk, 1), lambda r: (r, 0)),
            pl.BlockSpec((1, n), lambda r: (0, 0)),
            pl.BlockSpec((rblk, 1), lambda r: (r, 0)),
            pl.BlockSpec((1, n), lambda r: (0, 0)),
            pl.BlockSpec((rblk, hid), lambda r: (r, 0)),
            pl.BlockSpec((hid, n), lambda r: (0, 0)),
        ],
        out_specs=[
            pl.BlockSpec((hid, n), lambda r: (0, 0)),
            pl.BlockSpec((rblk, n), lambda r: (r, 0)),
            pl.BlockSpec((n, hid), lambda r: (0, 0)),
        ],
        out_shape=[
            jax.ShapeDtypeStruct((hid, n), f32),
            jax.ShapeDtypeStruct((n, n), jnp.bfloat16),
            jax.ShapeDtypeStruct((n, hid), f32),
        ],
        compiler_params=pltpu.CompilerParams(
            dimension_semantics=("arbitrary",)),
    )

    fa2_fn = pl.pallas_call(
        _fa2_kernel,
        grid=(nrb,),
        in_specs=[
            pl.BlockSpec((rblk, n), lambda r: (r, 0)),
            pl.BlockSpec((rblk, 1), lambda r: (r, 0)),
            pl.BlockSpec((1, n), lambda r: (0, 0)),
            pl.BlockSpec((rblk, hid), lambda r: (r, 0)),
            pl.BlockSpec((hid, n), lambda r: (0, 0)),
        ],
        out_specs=[
            pl.BlockSpec((hid, n), lambda r: (0, 0)),
            pl.BlockSpec((n, hid), lambda r: (0, 0)),
        ],
        out_shape=[
            jax.ShapeDtypeStruct((hid, n), f32),
            jax.ShapeDtypeStruct((n, hid), f32),
        ],
        compiler_params=pltpu.CompilerParams(
            dimension_semantics=("arbitrary",)),
    )

    raw_t = x0.T
    ta, tb = gate_fn(x0, gate_w_0, gate_b_0.reshape(1, 1))
    _, m, x1 = fa1_fn(code, ta, tb.reshape(1, n), ndh_r, ndh_c, ndt_r,
                      ndt_c, x0, raw_t)
    ta, tb = gate_fn(x1, gate_w_1, gate_b_1.reshape(1, 1))
    _, x = fa2_fn(m, ta, tb.reshape(1, n), x1, raw_t)

    out = pl.pallas_call(
        _head_kernel,
        grid=(nb,),
        in_specs=[
            pl.BlockSpec((blk, hid), lambda i: (i, 0)),
            pl.BlockSpec((cls, hid), lambda i: (0, 0)),
            pl.BlockSpec((1, cls), lambda i: (0, 0)),
        ],
        out_specs=pl.BlockSpec((blk, cls), lambda i: (i, 0)),
        out_shape=jax.ShapeDtypeStruct((n, cls), f32),
    )(x, t2_w, t2_b.reshape(1, cls))

    return out


# speedup vs baseline: 2.0105x; 1.0225x over previous
import jax
import jax.numpy as jnp
from jax.experimental import pallas as pl
from jax.experimental.pallas import tpu as pltpu

EPS = 0.3


def _relu_linear_kernel(h_ref, w_ref, b_ref, o_ref):
    o_ref[...] = jax.nn.relu(
        jax.lax.dot_general(h_ref[...], w_ref[...], (((1,), (1,)), ((), ())),
                            preferred_element_type=jnp.float32) + b_ref[...])


def _nd(d):
    return jnp.where(d > 0, jax.lax.rsqrt(d), 0.0)


def _pack_kernel(ah_ref, at_ref, code_ref, dh_ref, dt_ref):
    r = pl.program_id(0)
    nr = pl.num_programs(0)
    ah = ah_ref[...]
    at = at_ref[...]
    code_ref[...] = (ah + 2.0 * at).astype(jnp.int8)

    @pl.when(r == 0)
    def _():
        dh_ref[...] = jnp.zeros_like(dh_ref)
        dt_ref[...] = jnp.zeros_like(dt_ref)

    dh_ref[...] += jnp.sum(ah, axis=0, keepdims=True)
    dt_ref[...] += jnp.sum(at, axis=0, keepdims=True)

    @pl.when(r == nr - 1)
    def _():
        dh_ref[...] = _nd(dh_ref[...])
        dt_ref[...] = _nd(dt_ref[...])


def _gate_kernel(x_ref, gw_ref, gb_ref, ta_ref, tb_ref):
    gw = gw_ref[...]
    hid = x_ref.shape[1]
    gwa = gw[:, :hid]
    gwb = gw[:, hid:]
    x = x_ref[...]
    a = jnp.sum(x * gwa, axis=1, keepdims=True)
    b = jnp.sum(x * gwb, axis=1, keepdims=True)
    ta_ref[...] = a + gb_ref[0, 0]
    tb_ref[...] = b


def _gate_t(ta_ref, tb_ref):
    ta = ta_ref[...].astype(jnp.bfloat16)
    tb = tb_ref[...].astype(jnp.bfloat16)
    return jnp.tanh(ta + tb)


def _fa_matmul(w, x_ref, raw_ref, o_ref, on_ref, r, nr):
    xt = x_ref[...].T.astype(jnp.bfloat16)
    p = jax.lax.dot_general(xt, w, (((1,), (0,)), ((), ())),
                            preferred_element_type=jnp.float32)

    @pl.when(r == 0)
    def _():
        o_ref[...] = EPS * raw_ref[...]

    o_ref[...] += p

    @pl.when(r == nr - 1)
    def _():
        on_ref[...] = o_ref[...].T


def _fa1_kernel(code_ref, ta_ref, tb_ref, ndhr_ref, ndhc_ref, ndtr_ref,
                ndtc_ref, x_ref, raw_ref, gw2_ref, gb2_ref, o_ref, m_ref,
                on_ref, ta2_ref, tb2_ref):
    bf = jnp.bfloat16
    t = _gate_t(ta_ref, tb_ref)
    wh = ((0.5 * ndhr_ref[...]).astype(bf)) * ndhc_ref[...].astype(bf)
    wt = ((0.5 * ndtr_ref[...]).astype(bf)) * ndtc_ref[...].astype(bf)
    codef = code_ref[...].astype(bf)
    at = jnp.where(codef >= 2, jnp.ones((), bf), jnp.zeros((), bf))
    ah = codef - (at + at)
    m = ah * wh + at * wt
    m_ref[...] = m
    w = t * m
    r = pl.program_id(0)
    nr = pl.num_programs(0)
    _fa_matmul(w, x_ref, raw_ref, o_ref, on_ref, r, nr)

    @pl.when(r == nr - 1)
    def _():
        gw = gw2_ref[...]
        hid = x_ref.shape[1]
        x1 = on_ref[...]
        ta2_ref[...] = jnp.sum(x1 * gw[:, :hid], axis=1,
                               keepdims=True) + gb2_ref[0, 0]
        tb2_ref[...] = jnp.sum(x1 * gw[:, hid:], axis=1, keepdims=True)


def _fa2_kernel(m_ref, ta_ref, tb_ref, x_ref, raw_ref, w2_ref, b2_ref, o_ref,
                out_ref):
    t = _gate_t(ta_ref, tb_ref)
    w = t * m_ref[...]
    xt = x_ref[...].T.astype(jnp.bfloat16)
    p = jax.lax.dot_general(xt, w, (((1,), (0,)), ((), ())),
                            preferred_element_type=jnp.float32)
    r = pl.program_id(0)

    @pl.when(r == 0)
    def _():
        o_ref[...] = EPS * raw_ref[...]

    o_ref[...] += p

    @pl.when(r == pl.num_programs(0) - 1)
    def _():
        lt = jax.lax.dot_general(w2_ref[...], o_ref[...],
                                 (((1,), (0,)), ((), ())),
                                 preferred_element_type=jnp.float32)
        lt = lt + b2_ref[...]
        mx = jnp.max(lt, axis=0, keepdims=True)
        lsm = lt - mx - jnp.log(jnp.sum(jnp.exp(lt - mx), axis=0,
                                        keepdims=True))
        out_ref[...] = lsm.T


def _head_kernel(x_ref, w_ref, b_ref, o_ref):
    l = jax.lax.dot_general(x_ref[...], w_ref[...], (((1,), (1,)), ((), ())),
                            preferred_element_type=jnp.float32) + b_ref[...]
    m = jnp.max(l, axis=1, keepdims=True)
    o_ref[...] = l - m - jnp.log(jnp.sum(jnp.exp(l - m), axis=1, keepdims=True))


def kernel(h, adj_hom, adj_het, t1_w, t1_b, gate_w_0, gate_b_0, gate_w_1,
           gate_b_1, t2_w, t2_b):
    n, feat = h.shape
    hid = t1_w.shape[0]
    cls = t2_w.shape[0]
    f32 = jnp.float32

    blk = 1000 if n % 1000 == 0 else n
    nb = n // blk
    pblk = 200 if n % 200 == 0 else n
    npb = n // pblk
    rblk = 400 if n % 400 == 0 else n
    nrb = n // rblk

    x0 = pl.pallas_call(
        _relu_linear_kernel,
        grid=(nb,),
        in_specs=[
            pl.BlockSpec((blk, feat), lambda i: (i, 0)),
            pl.BlockSpec((hid, feat), lambda i: (0, 0)),
            pl.BlockSpec((1, hid), lambda i: (0, 0)),
        ],
        out_specs=pl.BlockSpec((blk, hid), lambda i: (i, 0)),
        out_shape=jax.ShapeDtypeStruct((n, hid), f32),
    )(h, t1_w, t1_b.reshape(1, hid))

    code, ndh, ndt = pl.pallas_call(
        _pack_kernel,
        grid=(npb,),
        in_specs=[
            pl.BlockSpec((pblk, n), lambda r: (r, 0)),
            pl.BlockSpec((pblk, n), lambda r: (r, 0)),
        ],
        out_specs=[
            pl.BlockSpec((pblk, n), lambda r: (r, 0)),
            pl.BlockSpec((1, n), lambda r: (0, 0)),
            pl.BlockSpec((1, n), lambda r: (0, 0)),
        ],
        out_shape=[
            jax.ShapeDtypeStruct((n, n), jnp.int8),
            jax.ShapeDtypeStruct((1, n), f32),
            jax.ShapeDtypeStruct((1, n), f32),
        ],
        compiler_params=pltpu.CompilerParams(
            dimension_semantics=("arbitrary",)),
    )(adj_hom, adj_het)

    ndh_c = ndh
    ndt_c = ndt
    ndh_r = ndh.reshape(n, 1)
    ndt_r = ndt.reshape(n, 1)

    gate_fn = pl.pallas_call(
        _gate_kernel,
        grid=(nb,),
        in_specs=[
            pl.BlockSpec((blk, hid), lambda i: (i, 0)),
            pl.BlockSpec((1, 2 * hid), lambda i: (0, 0)),
            pl.BlockSpec((1, 1), lambda i: (0, 0)),
        ],
        out_specs=[
            pl.BlockSpec((blk, 1), lambda i: (i, 0)),
            pl.BlockSpec((blk, 1), lambda i: (i, 0)),
        ],
        out_shape=[
            jax.ShapeDtypeStruct((n, 1), f32),
            jax.ShapeDtypeStruct((n, 1), f32),
        ],
    )

    fa1_fn = pl.pallas_call(
        _fa1_kernel,
        grid=(nrb,),
        in_specs=[
            pl.BlockSpec((rblk, n), lambda r: (r, 0)),
            pl.BlockSpec((rblk, 1), lambda r: (r, 0)),
            pl.BlockSpec((1, n), lambda r: (0, 0)),
            pl.BlockSpec((rblk, 1), lambda r: (r, 0)),
            pl.BlockSpec((1, n), lambda r: (0, 0)),
            pl.BlockSpec((rblk, 1), lambda r: (r, 0)),
            pl.BlockSpec((1, n), lambda r: (0, 0)),
            pl.BlockSpec((rblk, hid), lambda r: (r, 0)),
            pl.BlockSpec((hid, n), lambda r: (0, 0)),
            pl.BlockSpec((1, 2 * hid), lambda r: (0, 0)),
            pl.BlockSpec((1, 1), lambda r: (0, 0)),
        ],
        out_specs=[
            pl.BlockSpec((hid, n), lambda r: (0, 0)),
            pl.BlockSpec((rblk, n), lambda r: (r, 0)),
            pl.BlockSpec((n, hid), lambda r: (0, 0)),
            pl.BlockSpec((n, 1), lambda r: (0, 0)),
            pl.BlockSpec((n, 1), lambda r: (0, 0)),
        ],
        out_shape=[
            jax.ShapeDtypeStruct((hid, n), f32),
            jax.ShapeDtypeStruct((n, n), jnp.bfloat16),
            jax.ShapeDtypeStruct((n, hid), f32),
            jax.ShapeDtypeStruct((n, 1), f32),
            jax.ShapeDtypeStruct((n, 1), f32),
        ],
        compiler_params=pltpu.CompilerParams(
            dimension_semantics=("arbitrary",)),
    )

    fa2_fn = pl.pallas_call(
        _fa2_kernel,
        grid=(nrb,),
        in_specs=[
            pl.BlockSpec((rblk, n), lambda r: (r, 0)),
            pl.BlockSpec((rblk, 1), lambda r: (r, 0)),
            pl.BlockSpec((1, n), lambda r: (0, 0)),
            pl.BlockSpec((rblk, hid), lambda r: (r, 0)),
            pl.BlockSpec((hid, n), lambda r: (0, 0)),
            pl.BlockSpec((cls, hid), lambda r: (0, 0)),
            pl.BlockSpec((cls, 1), lambda r: (0, 0)),
        ],
        out_specs=[
            pl.BlockSpec((hid, n), lambda r: (0, 0)),
            pl.BlockSpec((n, cls), lambda r: (0, 0)),
        ],
        out_shape=[
            jax.ShapeDtypeStruct((hid, n), f32),
            jax.ShapeDtypeStruct((n, cls), f32),
        ],
        compiler_params=pltpu.CompilerParams(
            dimension_semantics=("arbitrary",)),
    )

    raw_t = x0.T
    ta, tb = gate_fn(x0, gate_w_0, gate_b_0.reshape(1, 1))
    _, m, x1, ta2, tb2 = fa1_fn(code, ta, tb.reshape(1, n), ndh_r, ndh_c,
                                ndt_r, ndt_c, x0, raw_t, gate_w_1,
                                gate_b_1.reshape(1, 1))
    _, out = fa2_fn(m, ta2, tb2.reshape(1, n), x1, raw_t, t2_w,
                    t2_b.reshape(cls, 1))

    return out


# speedup vs baseline: 2.0972x; 1.0431x over previous
import jax
import jax.numpy as jnp
from jax.experimental import pallas as pl
from jax.experimental.pallas import tpu as pltpu

EPS = 0.3


def _nd(d):
    return jnp.where(d > 0, jax.lax.rsqrt(d), 0.0)


def _pack_kernel(ah_ref, at_ref, code_ref, dh_ref, dt_ref):
    r = pl.program_id(0)
    nr = pl.num_programs(0)
    ah = ah_ref[...]
    at = at_ref[...]
    code_ref[...] = (ah + 2.0 * at).astype(jnp.int8)

    @pl.when(r == 0)
    def _():
        dh_ref[...] = jnp.zeros_like(dh_ref)
        dt_ref[...] = jnp.zeros_like(dt_ref)

    dh_ref[...] += jnp.sum(ah, axis=0, keepdims=True)
    dt_ref[...] += jnp.sum(at, axis=0, keepdims=True)

    @pl.when(r == nr - 1)
    def _():
        dh_ref[...] = _nd(dh_ref[...])
        dt_ref[...] = _nd(dt_ref[...])


def _relu_linear_kernel(h_ref, w_ref, b_ref, gw_ref, gb_ref, o_ref, ot_ref,
                        ta_ref, tb_ref):
    x0 = jax.nn.relu(
        jax.lax.dot_general(h_ref[...], w_ref[...], (((1,), (1,)), ((), ())),
                            preferred_element_type=jnp.float32) + b_ref[...])
    o_ref[...] = x0
    x0t = x0.T
    ot_ref[...] = x0t
    gw = gw_ref[...]
    hid = w_ref.shape[0]
    ta_ref[...] = jnp.sum(x0t * gw[:hid, :], axis=0,
                          keepdims=True) + gb_ref[0, 0]
    tb_ref[...] = jnp.sum(x0t * gw[hid:, :], axis=0, keepdims=True)


def _gate_t(ta_ref, tb_ref):
    ta = ta_ref[...].astype(jnp.bfloat16)
    tb = tb_ref[...].astype(jnp.bfloat16)
    return jnp.tanh(ta + tb)


def _fa_matmul(w, x_ref, raw_ref, o_ref, on_ref, r, nr):
    xt = x_ref[...].T.astype(jnp.bfloat16)
    p = jax.lax.dot_general(xt, w, (((1,), (0,)), ((), ())),
                            preferred_element_type=jnp.float32)

    @pl.when(r == 0)
    def _():
        o_ref[...] = EPS * raw_ref[...]

    o_ref[...] += p

    @pl.when(r == nr - 1)
    def _():
        on_ref[...] = o_ref[...].T


def _fa1_kernel(code_ref, ta_ref, tb_ref, ndhr_ref, ndhc_ref, ndtr_ref,
                ndtc_ref, x_ref, raw_ref, gw2_ref, gb2_ref, o_ref, m_ref,
                on_ref, ta2_ref, tb2_ref):
    bf = jnp.bfloat16
    t = _gate_t(ta_ref, tb_ref)
    wh = ((0.5 * ndhr_ref[...]).astype(bf)) * ndhc_ref[...].astype(bf)
    wt = ((0.5 * ndtr_ref[...]).astype(bf)) * ndtc_ref[...].astype(bf)
    codef = code_ref[...].astype(bf)
    at = jnp.where(codef >= 2, jnp.ones((), bf), jnp.zeros((), bf))
    ah = codef - (at + at)
    m = ah * wh + at * wt
    m_ref[...] = m
    w = t * m
    r = pl.program_id(0)
    nr = pl.num_programs(0)
    _fa_matmul(w, x_ref, raw_ref, o_ref, on_ref, r, nr)

    @pl.when(r == nr - 1)
    def _():
        gw = gw2_ref[...]
        hid = x_ref.shape[1]
        x1t = o_ref[...]
        ta2_ref[...] = jnp.sum(x1t * gw[:hid, :], axis=0,
                               keepdims=True) + gb2_ref[0, 0]
        tb2_ref[...] = jnp.sum(x1t * gw[hid:, :], axis=0, keepdims=True)


def _fa2_kernel(m_ref, ta_ref, tb_ref, x_ref, raw_ref, w2_ref, b2_ref, o_ref,
                out_ref):
    t = _gate_t(ta_ref, tb_ref)
    w = t * m_ref[...]
    xt = x_ref[...].T.astype(jnp.bfloat16)
    p = jax.lax.dot_general(xt, w, (((1,), (0,)), ((), ())),
                            preferred_element_type=jnp.float32)
    r = pl.program_id(0)

    @pl.when(r == 0)
    def _():
        o_ref[...] = EPS * raw_ref[...]

    o_ref[...] += p

    @pl.when(r == pl.num_programs(0) - 1)
    def _():
        lt = jax.lax.dot_general(w2_ref[...], o_ref[...],
                                 (((1,), (0,)), ((), ())),
                                 preferred_element_type=jnp.float32)
        lt = lt + b2_ref[...]
        mx = jnp.max(lt, axis=0, keepdims=True)
        lsm = lt - mx - jnp.log(jnp.sum(jnp.exp(lt - mx), axis=0,
                                        keepdims=True))
        out_ref[...] = lsm.T


def kernel(h, adj_hom, adj_het, t1_w, t1_b, gate_w_0, gate_b_0, gate_w_1,
           gate_b_1, t2_w, t2_b):
    n, feat = h.shape
    hid = t1_w.shape[0]
    cls = t2_w.shape[0]
    f32 = jnp.float32

    blk = 1000 if n % 1000 == 0 else n
    nb = n // blk
    pblk = 200 if n % 200 == 0 else n
    npb = n // pblk
    rblk = 400 if n % 400 == 0 else n
    nrb = n // rblk

    code, ndh, ndt = pl.pallas_call(
        _pack_kernel,
        grid=(npb,),
        in_specs=[
            pl.BlockSpec((pblk, n), lambda r: (r, 0)),
            pl.BlockSpec((pblk, n), lambda r: (r, 0)),
        ],
        out_specs=[
            pl.BlockSpec((pblk, n), lambda r: (r, 0)),
            pl.BlockSpec((1, n), lambda r: (0, 0)),
            pl.BlockSpec((1, n), lambda r: (0, 0)),
        ],
        out_shape=[
            jax.ShapeDtypeStruct((n, n), jnp.int8),
            jax.ShapeDtypeStruct((1, n), f32),
            jax.ShapeDtypeStruct((1, n), f32),
        ],
        compiler_params=pltpu.CompilerParams(
            dimension_semantics=("arbitrary",)),
    )(adj_hom, adj_het)

    x0, raw_t, ta, tb = pl.pallas_call(
        _relu_linear_kernel,
        grid=(1,),
        in_specs=[
            pl.BlockSpec((n, feat), lambda i: (0, 0)),
            pl.BlockSpec((hid, feat), lambda i: (0, 0)),
            pl.BlockSpec((1, hid), lambda i: (0, 0)),
            pl.BlockSpec((2 * hid, 1), lambda i: (0, 0)),
            pl.BlockSpec((1, 1), lambda i: (0, 0)),
        ],
        out_specs=[
            pl.BlockSpec((n, hid), lambda i: (0, 0)),
            pl.BlockSpec((hid, n), lambda i: (0, 0)),
            pl.BlockSpec((1, n), lambda i: (0, 0)),
            pl.BlockSpec((1, n), lambda i: (0, 0)),
        ],
        out_shape=[
            jax.ShapeDtypeStruct((n, hid), f32),
            jax.ShapeDtypeStruct((hid, n), f32),
            jax.ShapeDtypeStruct((1, n), f32),
            jax.ShapeDtypeStruct((1, n), f32),
        ],
    )(h, t1_w, t1_b.reshape(1, hid), gate_w_0.reshape(2 * hid, 1),
      gate_b_0.reshape(1, 1))

    ndh_c = ndh
    ndt_c = ndt
    ndh_r = ndh.reshape(n, 1)
    ndt_r = ndt.reshape(n, 1)

    fa1_fn = pl.pallas_call(
        _fa1_kernel,
        grid=(nrb,),
        in_specs=[
            pl.BlockSpec((rblk, n), lambda r: (r, 0)),
            pl.BlockSpec((rblk, 1), lambda r: (r, 0)),
            pl.BlockSpec((1, n), lambda r: (0, 0)),
            pl.BlockSpec((rblk, 1), lambda r: (r, 0)),
            pl.BlockSpec((1, n), lambda r: (0, 0)),
            pl.BlockSpec((rblk, 1), lambda r: (r, 0)),
            pl.BlockSpec((1, n), lambda r: (0, 0)),
            pl.BlockSpec((rblk, hid), lambda r: (r, 0)),
            pl.BlockSpec((hid, n), lambda r: (0, 0)),
            pl.BlockSpec((2 * hid, 1), lambda r: (0, 0)),
            pl.BlockSpec((1, 1), lambda r: (0, 0)),
        ],
        out_specs=[
            pl.BlockSpec((hid, n), lambda r: (0, 0)),
            pl.BlockSpec((rblk, n), lambda r: (r, 0)),
            pl.BlockSpec((n, hid), lambda r: (0, 0)),
            pl.BlockSpec((1, n), lambda r: (0, 0)),
            pl.BlockSpec((1, n), lambda r: (0, 0)),
        ],
        out_shape=[
            jax.ShapeDtypeStruct((hid, n), f32),
            jax.ShapeDtypeStruct((n, n), jnp.bfloat16),
            jax.ShapeDtypeStruct((n, hid), f32),
            jax.ShapeDtypeStruct((1, n), f32),
            jax.ShapeDtypeStruct((1, n), f32),
        ],
        compiler_params=pltpu.CompilerParams(
            dimension_semantics=("arbitrary",)),
    )

    fa2_fn = pl.pallas_call(
        _fa2_kernel,
        grid=(nrb,),
        in_specs=[
            pl.BlockSpec((rblk, n), lambda r: (r, 0)),
            pl.BlockSpec((rblk, 1), lambda r: (r, 0)),
            pl.BlockSpec((1, n), lambda r: (0, 0)),
            pl.BlockSpec((rblk, hid), lambda r: (r, 0)),
            pl.BlockSpec((hid, n), lambda r: (0, 0)),
            pl.BlockSpec((cls, hid), lambda r: (0, 0)),
            pl.BlockSpec((cls, 1), lambda r: (0, 0)),
        ],
        out_specs=[
            pl.BlockSpec((hid, n), lambda r: (0, 0)),
            pl.BlockSpec((n, cls), lambda r: (0, 0)),
        ],
        out_shape=[
            jax.ShapeDtypeStruct((hid, n), f32),
            jax.ShapeDtypeStruct((n, cls), f32),
        ],
        compiler_params=pltpu.CompilerParams(
            dimension_semantics=("arbitrary",)),
    )

    _, m, x1, ta2, tb2 = fa1_fn(code, ta.reshape(n, 1), tb, ndh_r, ndh_c,
                                ndt_r, ndt_c, x0, raw_t,
                                gate_w_1.reshape(2 * hid, 1),
                                gate_b_1.reshape(1, 1))
    _, out = fa2_fn(m, ta2.reshape(n, 1), tb2, x1, raw_t, t2_w,
                    t2_b.reshape(cls, 1))

    return out


# speedup vs baseline: 2.0978x; 1.0003x over previous
import jax
import jax.numpy as jnp
from jax.experimental import pallas as pl
from jax.experimental.pallas import tpu as pltpu

EPS = 0.3


def _nd(d):
    return jnp.where(d > 0, jax.lax.rsqrt(d), 0.0)


def _pack_kernel(ah_ref, at_ref, code_ref, dh_ref, dt_ref):
    r = pl.program_id(0)
    nr = pl.num_programs(0)
    ah = ah_ref[...]
    at = at_ref[...]
    code_ref[...] = (ah + 2.0 * at).astype(jnp.int8)

    @pl.when(r == 0)
    def _():
        dh_ref[...] = jnp.zeros_like(dh_ref)
        dt_ref[...] = jnp.zeros_like(dt_ref)

    dh_ref[...] += jnp.sum(ah, axis=0, keepdims=True)
    dt_ref[...] += jnp.sum(at, axis=0, keepdims=True)

    @pl.when(r == nr - 1)
    def _():
        dh_ref[...] = _nd(dh_ref[...])
        dt_ref[...] = _nd(dt_ref[...])


def _relu_linear_kernel(h_ref, w_ref, b_ref, gw_ref, gb_ref, o_ref, ot_ref,
                        ta_ref, tb_ref):
    x0 = jax.nn.relu(
        jax.lax.dot_general(h_ref[...], w_ref[...], (((1,), (1,)), ((), ())),
                            preferred_element_type=jnp.float32) + b_ref[...])
    o_ref[...] = x0
    x0t = x0.T
    ot_ref[...] = x0t
    gw = gw_ref[...]
    hid = w_ref.shape[0]
    ta_ref[...] = jnp.sum(x0t * gw[:hid, :], axis=0,
                          keepdims=True) + gb_ref[0, 0]
    tb_ref[...] = jnp.sum(x0t * gw[hid:, :], axis=0, keepdims=True)


def _gate_t(ta_ref, tb_ref):
    ta = ta_ref[...].astype(jnp.bfloat16)
    tb = tb_ref[...].astype(jnp.bfloat16)
    return jnp.tanh(ta + tb)


def _fa_matmul(w, x_ref, raw_ref, o_ref, on_ref, r, nr):
    xt = x_ref[...].T.astype(jnp.bfloat16)
    p = jax.lax.dot_general(xt, w, (((1,), (0,)), ((), ())),
                            preferred_element_type=jnp.float32)

    @pl.when(r == 0)
    def _():
        o_ref[...] = EPS * raw_ref[...]

    o_ref[...] += p

    @pl.when(r == nr - 1)
    def _():
        on_ref[...] = o_ref[...].T


def _fa1_kernel(code_ref, ta_ref, tb_ref, ndhr_ref, ndhc_ref, ndtr_ref,
                ndtc_ref, x_ref, raw_ref, gw2_ref, gb2_ref, o_ref, m_ref,
                on_ref, ta2_ref, tb2_ref):
    bf = jnp.bfloat16
    t = _gate_t(ta_ref, tb_ref)
    wh = ((0.5 * ndhr_ref[...]).astype(bf)) * ndhc_ref[...].astype(bf)
    wt = ((0.5 * ndtr_ref[...]).astype(bf)) * ndtc_ref[...].astype(bf)
    codef = code_ref[...].astype(bf)
    at = jnp.where(codef >= 2, jnp.ones((), bf), jnp.zeros((), bf))
    ah = codef - (at + at)
    m = ah * wh + at * wt
    m_ref[...] = m
    w = t * m
    r = pl.program_id(0)
    nr = pl.num_programs(0)
    _fa_matmul(w, x_ref, raw_ref, o_ref, on_ref, r, nr)

    @pl.when(r == nr - 1)
    def _():
        gw = gw2_ref[...]
        hid = x_ref.shape[1]
        x1t = o_ref[...]
        ta2_ref[...] = jnp.sum(x1t * gw[:hid, :], axis=0,
                               keepdims=True) + gb2_ref[0, 0]
        tb2_ref[...] = jnp.sum(x1t * gw[hid:, :], axis=0, keepdims=True)


def _fa2_kernel(m_ref, ta_ref, tb_ref, x_ref, raw_ref, w2_ref, b2_ref, o_ref,
                out_ref):
    t = _gate_t(ta_ref, tb_ref)
    w = t * m_ref[...]
    xt = x_ref[...].T.astype(jnp.bfloat16)
    p = jax.lax.dot_general(xt, w, (((1,), (0,)), ((), ())),
                            preferred_element_type=jnp.float32)
    r = pl.program_id(0)

    @pl.when(r == 0)
    def _():
        o_ref[...] = EPS * raw_ref[...]

    o_ref[...] += p

    @pl.when(r == pl.num_programs(0) - 1)
    def _():
        lt = jax.lax.dot_general(w2_ref[...], o_ref[...],
                                 (((1,), (0,)), ((), ())),
                                 preferred_element_type=jnp.float32)
        lt = lt + b2_ref[...]
        mx = jnp.max(lt, axis=0, keepdims=True)
        lsm = lt - mx - jnp.log(jnp.sum(jnp.exp(lt - mx), axis=0,
                                        keepdims=True))
        out_ref[...] = lsm.T


def kernel(h, adj_hom, adj_het, t1_w, t1_b, gate_w_0, gate_b_0, gate_w_1,
           gate_b_1, t2_w, t2_b):
    n, feat = h.shape
    hid = t1_w.shape[0]
    cls = t2_w.shape[0]
    f32 = jnp.float32

    pblk = 200 if n % 200 == 0 else n
    npb = n // pblk
    rblk = 400 if n % 400 == 0 else n
    nrb = n // rblk
    rblk2 = 400 if n % 400 == 0 else n
    nrb2 = n // rblk2

    code, ndh, ndt = pl.pallas_call(
        _pack_kernel,
        grid=(npb,),
        in_specs=[
            pl.BlockSpec((pblk, n), lambda r: (r, 0)),
            pl.BlockSpec((pblk, n), lambda r: (r, 0)),
        ],
        out_specs=[
            pl.BlockSpec((pblk, n), lambda r: (r, 0)),
            pl.BlockSpec((1, n), lambda r: (0, 0)),
            pl.BlockSpec((1, n), lambda r: (0, 0)),
        ],
        out_shape=[
            jax.ShapeDtypeStruct((n, n), jnp.int8),
            jax.ShapeDtypeStruct((1, n), f32),
            jax.ShapeDtypeStruct((1, n), f32),
        ],
        compiler_params=pltpu.CompilerParams(
            dimension_semantics=("arbitrary",)),
    )(adj_hom, adj_het)

    x0, raw_t, ta, tb = pl.pallas_call(
        _relu_linear_kernel,
        grid=(1,),
        in_specs=[
            pl.BlockSpec((n, feat), lambda i: (0, 0)),
            pl.BlockSpec((hid, feat), lambda i: (0, 0)),
            pl.BlockSpec((1, hid), lambda i: (0, 0)),
            pl.BlockSpec((2 * hid, 1), lambda i: (0, 0)),
            pl.BlockSpec((1, 1), lambda i: (0, 0)),
        ],
        out_specs=[
            pl.BlockSpec((n, hid), lambda i: (0, 0)),
            pl.BlockSpec((hid, n), lambda i: (0, 0)),
            pl.BlockSpec((1, n), lambda i: (0, 0)),
            pl.BlockSpec((1, n), lambda i: (0, 0)),
        ],
        out_shape=[
            jax.ShapeDtypeStruct((n, hid), f32),
            jax.ShapeDtypeStruct((hid, n), f32),
            jax.ShapeDtypeStruct((1, n), f32),
            jax.ShapeDtypeStruct((1, n), f32),
        ],
    )(h, t1_w, t1_b.reshape(1, hid), gate_w_0.reshape(2 * hid, 1),
      gate_b_0.reshape(1, 1))

    ndh_c = ndh
    ndt_c = ndt
    ndh_r = ndh.reshape(n, 1)
    ndt_r = ndt.reshape(n, 1)

    fa1_fn = pl.pallas_call(
        _fa1_kernel,
        grid=(nrb,),
        in_specs=[
            pl.BlockSpec((rblk, n), lambda r: (r, 0)),
            pl.BlockSpec((rblk, 1), lambda r: (r, 0)),
            pl.BlockSpec((1, n), lambda r: (0, 0)),
            pl.BlockSpec((rblk, 1), lambda r: (r, 0)),
            pl.BlockSpec((1, n), lambda r: (0, 0)),
            pl.BlockSpec((rblk, 1), lambda r: (r, 0)),
            pl.BlockSpec((1, n), lambda r: (0, 0)),
            pl.BlockSpec((rblk, hid), lambda r: (r, 0)),
            pl.BlockSpec((hid, n), lambda r: (0, 0)),
            pl.BlockSpec((2 * hid, 1), lambda r: (0, 0)),
            pl.BlockSpec((1, 1), lambda r: (0, 0)),
        ],
        out_specs=[
            pl.BlockSpec((hid, n), lambda r: (0, 0)),
            pl.BlockSpec((rblk, n), lambda r: (r, 0)),
            pl.BlockSpec((n, hid), lambda r: (0, 0)),
            pl.BlockSpec((1, n), lambda r: (0, 0)),
            pl.BlockSpec((1, n), lambda r: (0, 0)),
        ],
        out_shape=[
            jax.ShapeDtypeStruct((hid, n), f32),
            jax.ShapeDtypeStruct((n, n), jnp.bfloat16),
            jax.ShapeDtypeStruct((n, hid), f32),
            jax.ShapeDtypeStruct((1, n), f32),
            jax.ShapeDtypeStruct((1, n), f32),
        ],
        compiler_params=pltpu.CompilerParams(
            dimension_semantics=("arbitrary",)),
    )

    fa2_fn = pl.pallas_call(
        _fa2_kernel,
        grid=(nrb2,),
        in_specs=[
            pl.BlockSpec((rblk2, n), lambda r: (r, 0)),
            pl.BlockSpec((rblk2, 1), lambda r: (r, 0)),
            pl.BlockSpec((1, n), lambda r: (0, 0)),
            pl.BlockSpec((rblk2, hid), lambda r: (r, 0)),
            pl.BlockSpec((hid, n), lambda r: (0, 0)),
            pl.BlockSpec((cls, hid), lambda r: (0, 0)),
            pl.BlockSpec((cls, 1), lambda r: (0, 0)),
        ],
        out_specs=[
            pl.BlockSpec((hid, n), lambda r: (0, 0)),
            pl.BlockSpec((n, cls), lambda r: (0, 0)),
        ],
        out_shape=[
            jax.ShapeDtypeStruct((hid, n), f32),
            jax.ShapeDtypeStruct((n, cls), f32),
        ],
        compiler_params=pltpu.CompilerParams(
            dimension_semantics=("arbitrary",)),
    )

    _, m, x1, ta2, tb2 = fa1_fn(code, ta.reshape(n, 1), tb, ndh_r, ndh_c,
                                ndt_r, ndt_c, x0, raw_t,
                                gate_w_1.reshape(2 * hid, 1),
                                gate_b_1.reshape(1, 1))
    _, out = fa2_fn(m, ta2.reshape(n, 1), tb2, x1, raw_t, t2_w,
                    t2_b.reshape(cls, 1))

    return out
